# Initial kernel scaffold; baseline (speedup 1.0000x reference)
import jax
import jax.numpy as jnp
from jax.experimental import pallas as pl


def kernel(initial_position_indexes, destination_indexes, initial_position_table, destination_table):
    raise NotImplementedError("write your pallas kernel here")


# speedup vs baseline: 2.8002x; 2.8002x over previous
import functools

import jax
import jax.numpy as jnp
from jax import lax
from jax.experimental import pallas as pl
from jax.experimental.pallas import tpu as pltpu
from jax.experimental.pallas import tpu_sc as plsc

EMBED_DIM = 128
LANES = 16


def _make_sc_kernel(n_rows: int, chunk: int, num_workers: int):
    per_w = n_rows // num_workers
    n_chunks = per_w // chunk
    mesh = plsc.VectorSubcoreMesh(core_axis_name="c", subcore_axis_name="s")

    @functools.partial(
        pl.kernel,
        mesh=mesh,
        out_type=jax.ShapeDtypeStruct((n_rows, EMBED_DIM), jnp.float32),
        scratch_types=[
            pltpu.VMEM((chunk,), jnp.int32),
            pltpu.VMEM((chunk,), jnp.int32),
            pltpu.VMEM((chunk, EMBED_DIM), jnp.float32),
            pltpu.VMEM((chunk, EMBED_DIM), jnp.float32),
            pltpu.SemaphoreType.DMA,
            pltpu.SemaphoreType.DMA,
        ],
    )
    def sc_kernel(i1_hbm, i2_hbm, t1_hbm, t2_hbm, out_hbm,
                  idx1_v, idx2_v, rows1_v, rows2_v, sem1, sem2):
        wid = lax.axis_index("s") * 2 + lax.axis_index("c")
        base = wid * per_w

        def do_chunk(ci, _):
            off = base + ci * chunk
            pltpu.sync_copy(i1_hbm.at[pl.ds(off, chunk)], idx1_v)
            pltpu.sync_copy(i2_hbm.at[pl.ds(off, chunk)], idx2_v)
            cp1 = pltpu.async_copy(t1_hbm.at[idx1_v], rows1_v, sem1)
            cp2 = pltpu.async_copy(t2_hbm.at[idx2_v], rows2_v, sem2)
            cp1.wait()
            cp2.wait()

            def add_row(r, _):
                for c in range(EMBED_DIM // LANES):
                    sl = (r, pl.ds(c * LANES, LANES))
                    rows1_v[sl] = rows1_v[sl] + rows2_v[sl]
                return 0

            lax.fori_loop(0, chunk, add_row, 0, unroll=False)
            pltpu.sync_copy(rows1_v, out_hbm.at[pl.ds(off, chunk)])
            return 0

        lax.fori_loop(0, n_chunks, do_chunk, 0, unroll=False)

    return sc_kernel


def kernel(initial_position_indexes, destination_indexes,
           initial_position_table, destination_table):
    b, l = initial_position_indexes.shape
    n = b * l
    i1 = initial_position_indexes.reshape(n).astype(jnp.int32)
    i2 = destination_indexes.reshape(n).astype(jnp.int32)
    sc = _make_sc_kernel(n_rows=n, chunk=256, num_workers=32)
    out = sc(i1, i2, initial_position_table, destination_table)
    return out.reshape(b, l, EMBED_DIM)


# speedup vs baseline: 5.6836x; 2.0297x over previous
import functools

import jax
import jax.numpy as jnp
from jax import lax
from jax.experimental import pallas as pl
from jax.experimental.pallas import tpu as pltpu
from jax.experimental.pallas import tpu_sc as plsc

EMBED_DIM = 128
LANES = 16
VOCAB_ROWS = 65


def _combine_tables(t1, t2):
    def body(t1_ref, t2_ref, out_ref):
        out_ref[...] = t1_ref[...][:, None, :] + t2_ref[...][None, :, :]

    out = pl.pallas_call(
        body,
        out_shape=jax.ShapeDtypeStruct(
            (VOCAB_ROWS, VOCAB_ROWS, EMBED_DIM), jnp.float32),
    )(t1, t2)
    return out.reshape(VOCAB_ROWS * VOCAB_ROWS, EMBED_DIM)


def _make_sc_kernel(n_rows: int, chunk: int, nbuf: int, num_workers: int):
    per_w = n_rows // num_workers
    n_chunks = per_w // chunk
    assert n_chunks % nbuf == 0 and n_chunks >= 2 * nbuf
    mesh = plsc.VectorSubcoreMesh(core_axis_name="c", subcore_axis_name="s")

    @functools.partial(
        pl.kernel,
        mesh=mesh,
        out_type=jax.ShapeDtypeStruct((n_rows, EMBED_DIM), jnp.float32),
        scratch_types=[
            pltpu.VMEM((per_w,), jnp.int32),
            pltpu.VMEM((per_w,), jnp.int32),
            pltpu.VMEM((nbuf, chunk, EMBED_DIM), jnp.float32),
        ]
        + [pltpu.SemaphoreType.DMA] * (2 * nbuf),
    )
    def sc_kernel(i1_hbm, i2_hbm, t12_hbm, out_hbm,
                  pidx_v, i2_v, rows_v, *sems):
        gsem = sems[:nbuf]
        osem = sems[nbuf:]
        wid = lax.axis_index("s") * 2 + lax.axis_index("c")
        base = wid * per_w

        pltpu.sync_copy(i1_hbm.at[pl.ds(base, per_w)], pidx_v)
        pltpu.sync_copy(i2_hbm.at[pl.ds(base, per_w)], i2_v)

        def fuse(j, _):
            sl = pl.ds(j * LANES, LANES)
            pidx_v[sl] = pidx_v[sl] * VOCAB_ROWS + i2_v[sl]
            return 0

        lax.fori_loop(0, per_w // LANES, fuse, 0, unroll=False)

        def gather(g, b):
            pltpu.async_copy(
                t12_hbm.at[pidx_v.at[pl.ds(g * chunk, chunk)]],
                rows_v.at[b], gsem[b])

        def gather_wait(b):
            pltpu.make_async_copy(
                t12_hbm.at[pl.ds(0, chunk)], rows_v.at[b], gsem[b]).wait()

        for b in range(nbuf):
            gather(b, b)

        def ring(i, _):
            g0 = i * nbuf
            for b in range(nbuf):
                g = g0 + b
                gather_wait(b)
                pltpu.async_copy(
                    rows_v.at[b], out_hbm.at[pl.ds(base + g * chunk, chunk)],
                    osem[b])
                nxt = g + nbuf

                @pl.when(nxt < n_chunks)
                def _():
                    pltpu.make_async_copy(
                        rows_v.at[b], out_hbm.at[pl.ds(base, chunk)],
                        osem[b]).wait()
                    gather(nxt, b)
            return 0

        lax.fori_loop(0, n_chunks // nbuf, ring, 0, unroll=False)
        for b in range(nbuf):
            pltpu.make_async_copy(
                rows_v.at[b],
                out_hbm.at[pl.ds(base, chunk)], osem[b]).wait()

    return sc_kernel


def kernel(initial_position_indexes, destination_indexes,
           initial_position_table, destination_table):
    b, l = initial_position_indexes.shape
    n = b * l
    i1 = initial_position_indexes.reshape(n).astype(jnp.int32)
    i2 = destination_indexes.reshape(n).astype(jnp.int32)
    t12 = _combine_tables(initial_position_table, destination_table)
    sc = _make_sc_kernel(n_rows=n, chunk=128, nbuf=5, num_workers=32)
    out = sc(i1, i2, t12)
    return out.reshape(b, l, EMBED_DIM)
